# Initial kernel scaffold; baseline (speedup 1.0000x reference)
import jax
import jax.numpy as jnp
from jax.experimental import pallas as pl


def kernel(x, edge_index, params):
    raise NotImplementedError("write your pallas kernel here")


# speedup vs baseline: 9.7331x; 9.7331x over previous
import functools

import jax
import jax.numpy as jnp
from jax import lax
from jax.experimental import pallas as pl
from jax.experimental.pallas import tpu as pltpu
from jax.experimental.pallas import tpu_sc as plsc

N = 10000
E = 320000
DIMS = [128, 40, 30, 20, 10, 5, 1, 5, 10, 20, 30, 40, 50]
NPAD = 10240
RPS = NPAD // 16
NW = 32
EPW = E // NW
K = 80
NCHUNK = EPW // K

DP = [min(DIMS[i], DIMS[i + 1]) for i in range(12)]
PRE = [DIMS[i] <= DIMS[i + 1] for i in range(12)]
DPP = [max(8, -(-d // 8) * 8) for d in DP]


@functools.lru_cache(None)
def _make_prop(d):
    mesh = plsc.VectorSubcoreMesh(core_axis_name="c", subcore_axis_name="s")

    def body(p_hbm, src_hbm, dst_hbm, z_hbm, out_hbm, acc, sidx, didx, rows, sem):
        c = lax.axis_index("c")
        s = lax.axis_index("s")
        wid = c * 16 + s
        pltpu.sync_copy(z_hbm.at[pl.ds(s * RPS, RPS)], acc.at[pl.ds(s * RPS, RPS)])
        plsc.subcore_barrier()
        base0 = wid * EPW

        def step(j, carry):
            base = base0 + j * K
            pltpu.sync_copy(src_hbm.at[pl.ds(base, K)], sidx)
            pltpu.sync_copy(dst_hbm.at[pl.ds(base, K)], didx)
            pltpu.async_copy(p_hbm.at[sidx], rows, sem).wait()
            pltpu.sync_copy(rows, acc.at[didx], add=True)
            return carry

        lax.fori_loop(0, NCHUNK, step, 0)
        plsc.subcore_barrier()
        pltpu.sync_copy(acc.at[pl.ds(s * RPS, RPS)],
                        out_hbm.at[c].at[pl.ds(s * RPS, RPS)])

    return pl.kernel(
        body,
        out_type=jax.ShapeDtypeStruct((2, NPAD, d), jnp.float32),
        mesh=mesh,
        compiler_params=pltpu.CompilerParams(use_tc_tiling_on_sc=False),
        scratch_types=[
            pltpu.VMEM_SHARED((NPAD, d), jnp.float32),
            pltpu.VMEM((K,), jnp.int32),
            pltpu.VMEM((K,), jnp.int32),
            pltpu.VMEM((K, d), jnp.float32),
            pltpu.SemaphoreType.DMA,
        ],
    )


def _elu(z):
    return jnp.where(z > 0, z, jnp.exp(jnp.minimum(z, 0.0)) - 1.0)


def _bn(h, g, be):
    m = jnp.mean(h, axis=0, keepdims=True)
    v = jnp.mean((h - m) * (h - m), axis=0, keepdims=True)
    return (h - m) * lax.rsqrt(v + 1e-5) * g + be


def _first_body(degp_ref, x_ref, w_ref, dinv_ref, p_ref):
    deg = 1.0 + degp_ref[0, :N, 0:1] + degp_ref[1, :N, 0:1]
    dinv = lax.rsqrt(deg)
    dinv_ref[...] = dinv
    y = jnp.dot(x_ref[...], w_ref[...], preferred_element_type=jnp.float32,
            precision=lax.Precision.HIGHEST)
    p_ref[...] = y * dinv


_T_FIRST = pl.pallas_call(
    _first_body,
    out_shape=(
        jax.ShapeDtypeStruct((N, 1), jnp.float32),
        jax.ShapeDtypeStruct((N, DPP[0]), jnp.float32),
    ),
)


def _make_mid(i):
    last = i == 11

    def body(parts_ref, p_ref, dinv_ref, *rest):
        refs = list(rest)
        out_ref = refs.pop()
        dv = dinv_ref[...]
        agg = parts_ref[0, :N, :] + parts_ref[1, :N, :] + p_ref[...]
        agg = agg * dv
        if PRE[i]:
            w = refs.pop(0)
            z = jnp.dot(agg, w[...], preferred_element_type=jnp.float32,
                            precision=lax.Precision.HIGHEST)
        else:
            z = agg
        b = refs.pop(0)
        z = z + b[...]
        if last:
            out_ref[...] = z
            return
        g = refs.pop(0)
        be = refs.pop(0)
        h = _bn(_elu(z), g[...], be[...])
        if PRE[i + 1]:
            pn = h * dv
            pad = DPP[i + 1] - pn.shape[1]
            if pad:
                pn = jnp.concatenate(
                    [pn, jnp.zeros((N, pad), jnp.float32)], axis=1)
            out_ref[...] = pn
        else:
            wn = refs.pop(0)
            out_ref[...] = jnp.dot(h, wn[...], preferred_element_type=jnp.float32,
                                 precision=lax.Precision.HIGHEST) * dv

    dout = DIMS[-1] if last else DPP[i + 1]
    return pl.pallas_call(
        body,
        out_shape=jax.ShapeDtypeStruct((N, dout), jnp.float32),
    )


_T_MIDS = [_make_mid(i) for i in range(12)]


def _pad_to(a, rows, cols):
    return jnp.pad(a, ((0, rows - a.shape[0]), (0, cols - a.shape[1])))


def kernel(x, edge_index, params):
    src = edge_index[0]
    dst = edge_index[1]
    Ws, bs, gs, bes = params["W"], params["b"], params["g"], params["be"]
    zeros = {d: jnp.zeros((NPAD, d), jnp.float32) for d in sorted(set(DPP))}

    ones = jnp.ones((N, 8), jnp.float32)
    deg_parts = _make_prop(8)(ones, src, dst, zeros[8])
    dinv, p = _T_FIRST(deg_parts, x, Ws[0])

    for i in range(12):
        parts = _make_prop(DPP[i])(p, src, dst, zeros[DPP[i]])
        args = [parts, p, dinv]
        if PRE[i]:
            args.append(_pad_to(Ws[i], DPP[i], DIMS[i + 1]))
            bw = DIMS[i + 1]
        else:
            bw = DPP[i]
        args.append(_pad_to(bs[i].reshape(1, -1), 1, bw))
        if i < 11:
            args.append(_pad_to(gs[i].reshape(1, -1), 1, bw))
            args.append(_pad_to(bes[i].reshape(1, -1), 1, bw))
            if not PRE[i + 1]:
                args.append(_pad_to(Ws[i + 1], bw, DPP[i + 1]))
        p = _T_MIDS[i](*args)
    return p
